# Initial kernel scaffold; baseline (speedup 1.0000x reference)
import jax
import jax.numpy as jnp
from jax.experimental import pallas as pl


def kernel(z, embeddings):
    raise NotImplementedError("write your pallas kernel here")


# speedup vs baseline: 1.1204x; 1.1204x over previous
import functools

import jax
import jax.numpy as jnp
from jax import lax
from jax.experimental import pallas as pl
from jax.experimental.pallas import tpu as pltpu
from jax.experimental.pallas import tpu_sc as plsc

_K = 8192
_D = 256
_TM = 256
_CHUNKS = ((0, 2736), (2736, 5472), (5472, 8192))

_NC = 2
_NS = 16
_NW = _NC * _NS


def _argmin_body(z_ref, eb_ref, zn_ref, en_ref, out_ref):
    z = z_ref[...]
    zb = (2.0 * z).astype(jnp.bfloat16)
    mm = lax.dot_general(zb, eb_ref[...], (((1,), (1,)), ((), ())),
                         preferred_element_type=jnp.float32)
    d = (zn_ref[...] - mm) + en_ref[...]
    run_v = jnp.full((_TM, 1), jnp.inf, jnp.float32)
    run_i = jnp.zeros((_TM, 1), jnp.int32)
    for lo, hi in _CHUNKS:
        dw = d[:, lo:hi]
        cm = jnp.min(dw, axis=1, keepdims=True)
        iota = lax.broadcasted_iota(jnp.int32, (_TM, hi - lo), 1)
        ci = jnp.min(jnp.where(dw == cm, iota, hi - lo), axis=1,
                     keepdims=True) + lo
        pred = cm < run_v
        run_i = jnp.where(pred, ci, run_i)
        run_v = jnp.where(pred, cm, run_v).astype(jnp.bfloat16).astype(jnp.float32)
    out_ref[...] = run_i


def _argmin_call(zf, eb, zn, en):
    m = zf.shape[0]
    return pl.pallas_call(
        _argmin_body,
        grid=(m // _TM,),
        in_specs=[
            pl.BlockSpec((_TM, _D), lambda i: (i, 0)),
            pl.BlockSpec((_K, _D), lambda i: (0, 0)),
            pl.BlockSpec((_TM, 1), lambda i: (i, 0)),
            pl.BlockSpec((1, _K), lambda i: (0, 0)),
        ],
        out_specs=pl.BlockSpec((_TM, 1), lambda i: (i, 0)),
        out_shape=jax.ShapeDtypeStruct((m, 1), jnp.int32),
    )(zf, eb, zn, en)


@functools.cache
def _make_sc_gather(b, d):
    b_per_w = b // _NW
    mesh = plsc.VectorSubcoreMesh(core_axis_name="c", subcore_axis_name="s")

    @functools.partial(
        pl.kernel,
        mesh=mesh,
        out_type=jax.ShapeDtypeStruct((b, d), jnp.float32),
        scratch_types=[
            pltpu.VMEM((b_per_w,), jnp.int32),
            pltpu.VMEM((b_per_w, d), jnp.float32),
            pltpu.SemaphoreType.DMA,
        ],
    )
    def gather(table_hbm, idx_hbm, out_hbm, idx_v, rows_v, sem):
        wid = lax.axis_index("s") * _NC + lax.axis_index("c")
        base = wid * b_per_w
        pltpu.sync_copy(idx_hbm.at[pl.ds(base, b_per_w)], idx_v)
        pltpu.async_copy(table_hbm.at[idx_v], rows_v, sem).wait()
        pltpu.sync_copy(rows_v, out_hbm.at[pl.ds(base, b_per_w)])

    return gather


def kernel(z, embeddings):
    zf = z.reshape(-1, _D)
    zn = jnp.sum(zf ** 2, axis=1, keepdims=True)
    en = jnp.sum(embeddings ** 2, axis=1)[None, :]
    eb = embeddings.astype(jnp.bfloat16)
    idx = _argmin_call(zf, eb, zn, en).reshape(-1)
    q = _make_sc_gather(zf.shape[0], _D)(embeddings, idx)
    return q.reshape(z.shape)
